# baseline (speedup 1.0000x reference)
import jax
import jax.numpy as jnp
from jax.experimental import pallas as pl
from jax.experimental.pallas import tpu as pltpu
from jax.experimental.pallas import tpu_sc as plsc

D_MODEL = 1024
NUM_TILES = 32
CHUNK = 32


def _sc_gather(table, idx2d, n):
    per_tile = n // NUM_TILES
    mesh = plsc.VectorSubcoreMesh(core_axis_name="c", subcore_axis_name="s")

    @pl.kernel(
        out_type=jax.ShapeDtypeStruct((n, D_MODEL), table.dtype),
        mesh=mesh,
        scratch_types=[
            pltpu.VMEM((1, per_tile), jnp.int32),
            pltpu.VMEM((CHUNK, D_MODEL), jnp.float32),
            pltpu.SemaphoreType.DMA,
        ],
    )
    def gather_kernel(table_hbm, idx_hbm, out_hbm, idx_v, buf, sem):
        c = jax.lax.axis_index("c")
        s = jax.lax.axis_index("s")
        t = c * 16 + s
        pltpu.async_copy(idx_hbm.at[pl.ds(t, 1), :], idx_v, sem).wait()

        @pl.loop(0, per_tile // CHUNK)
        def _(j):
            rows = idx_v.at[0, pl.ds(j * CHUNK, CHUNK)]
            pltpu.sync_copy(table_hbm.at[rows], buf)
            base = t * per_tile + j * CHUNK
            pltpu.sync_copy(buf, out_hbm.at[pl.ds(base, CHUNK), :])

    return gather_kernel(table, idx2d)


def kernel(mask_bits, table):
    b, s = mask_bits.shape
    n = b * s
    idx = mask_bits.astype(jnp.int32).reshape(NUM_TILES, n // NUM_TILES)
    out = _sc_gather(table, idx, n)
    return out.reshape(b, s, D_MODEL)
